# baseline (speedup 1.0000x reference)
import functools
import math

import jax
import jax.numpy as jnp
from jax import lax
from jax.experimental import pallas as pl
from jax.experimental.pallas import tpu as pltpu
from jax.experimental.pallas import tpu_sc as plsc

N = 2048
DIM = 768
NHEADS = 12
HD = DIM // NHEADS
KNN = 16
PDIM = 16
FF = 4 * DIM
BLK = 256
GRID = N // BLK

NW = 32
RPW = N // NW


def _ln(x, scale, bias):
    m = jnp.mean(x, axis=-1, keepdims=True)
    xc = x - m
    v = jnp.mean(xc * xc, axis=-1, keepdims=True)
    return xc / jnp.sqrt(v + 1e-6) * scale + bias


def _pre_body(x_ref, w_ref, b_ref, s_ref, t_ref, pb_ref, pf_ref, c_ref,
              qkv_ref, arg_ref):
    xn = _ln(x_ref[...], s_ref[...], t_ref[...])
    qkv_ref[...] = (
        jnp.dot(xn, w_ref[...], preferred_element_type=jnp.float32) + b_ref[...]
    )

    c = c_ref[0, 0]
    pb = pb_ref[...]
    pf = pf_ref[...]
    nb = jnp.sum(pb * pb, axis=1, keepdims=True)
    ones_p = jnp.ones((1, PDIM), jnp.float32)
    nf = lax.dot_general(
        ones_p, pf * pf, (((1,), (1,)), ((), ())),
        precision=lax.Precision.HIGHEST,
        preferred_element_type=jnp.float32)
    g = lax.dot_general(
        pb, pf, (((1,), (1,)), ((), ())),
        precision=lax.Precision.HIGHEST,
        preferred_element_type=jnp.float32)
    diff = jnp.maximum(nb + nf - 2.0 * g, 0.0)
    den = jnp.maximum((1.0 - c * nb) * (1.0 - c * nf), 1e-8)
    arg_ref[...] = 1.0 + 2.0 * c * diff / den


def _sc_topk(arg_hbm, idx_hbm, val_hbm, row_v, i16_v, v16_v, *, first_row,
             rows_per_worker):
    wid = lax.axis_index("s") * 2 + lax.axis_index("c")
    base = first_row + wid * rows_per_worker
    lane = lax.iota(jnp.int32, 16)

    def _halfclean(a_k, a_i, b_k, b_i):
        m = b_k < a_k
        return jnp.where(m, b_k, a_k), jnp.where(m, b_i, a_i)

    def row_body(r, carry0):
        row = base + r
        pltpu.sync_copy(arg_hbm.at[pl.ds(row * N, N)], row_v)

        def chunk_body(c8, carry):
            best, bidx = carry
            ks, vs = [], []
            for u in range(8):
                cc = c8 * 8 + u
                d = row_v[pl.ds(cc * 16, 16)]
                ci = lane + cc * 16
                sk, si = plsc.sort_key_val(d, ci, descending=(u % 2 == 1))
                ks.append(sk)
                vs.append(si)
            for _ in range(2):
                nk, nv = [], []
                for p in range(len(ks) // 2):
                    lk, li = _halfclean(ks[2 * p], vs[2 * p],
                                        ks[2 * p + 1], vs[2 * p + 1])
                    sk, si = plsc.sort_key_val(lk, li,
                                               descending=(p % 2 == 1))
                    nk.append(sk)
                    nv.append(si)
                ks, vs = nk, nv
            lk, li = _halfclean(ks[0], vs[0], ks[1], vs[1])
            xk, xi = plsc.sort_key_val(lk, li, descending=True)
            nb, ni = _halfclean(best, bidx, xk, xi)
            rk, ri = plsc.sort_key_val(nb, ni)
            return rk, ri

        best0 = jnp.full((16,), jnp.inf, jnp.float32)
        bidx0 = jnp.zeros((16,), jnp.int32)
        best, bidx = lax.fori_loop(0, N // 128, chunk_body, (best0, bidx0))
        i16_v[...] = bidx
        v16_v[...] = best
        out_off = (row - first_row) * KNN
        pltpu.sync_copy(i16_v, idx_hbm.at[pl.ds(out_off, KNN)])
        pltpu.sync_copy(v16_v, val_hbm.at[pl.ds(out_off, KNN)])
        return carry0

    lax.fori_loop(0, rows_per_worker, row_body, 0)


def _attn_mlp_body(idx_ref, val_ref, q_ref, k_ref, v_ref, x_ref, wo_ref,
                   bo_ref, w1_ref, b1_ref, w2_ref, b2_ref, s2_ref, t2_ref,
                   c_ref, lt_ref, as_ref, o_ref):
    c = c_ref[0, 0]
    inv_tau = 1.0 / jnp.maximum(jnp.exp(lt_ref[0, 0]), 1e-8)
    a_scale = as_ref[0, 0]
    inv_sqrt_c = 1.0 / jnp.sqrt(c)

    s = jnp.maximum(val_ref[...], 1.0 + 1e-7)
    d = jnp.log(s + jnp.sqrt((s - 1.0) * (s + 1.0))) * inv_sqrt_c
    gvals = -d * inv_tau
    idx = idx_ref[...]
    iota = lax.broadcasted_iota(jnp.int32, (BLK, N), 1)
    geo = jnp.zeros((BLK, N), jnp.float32)
    for t in range(KNN):
        geo = jnp.where(iota == idx[:, t:t + 1], gvals[:, t:t + 1], geo)
    mask = geo < 0.0

    inv_sqrt_hd = 1.0 / math.sqrt(HD)
    ones8 = jnp.ones((N, 8), jnp.float32)
    acc = x_ref[...] + bo_ref[...]
    for h in range(NHEADS):
        qh = q_ref[:, h * HD:(h + 1) * HD]
        kh = k_ref[:, h * HD:(h + 1) * HD]
        vh = v_ref[:, h * HD:(h + 1) * HD]
        sco = lax.dot_general(
            qh, kh, (((1,), (1,)), ((), ())),
            preferred_element_type=jnp.float32) * inv_sqrt_hd
        sc = a_scale * jnp.tanh(sco + geo)
        w = jnp.where(mask, jnp.exp(sc), 0.0)
        denom = jnp.dot(w, ones8, preferred_element_type=jnp.float32)
        oh = jnp.dot(w, vh, preferred_element_type=jnp.float32)
        oh = oh * (1.0 / denom[:, 0:1])
        acc = acc + jnp.dot(
            oh, wo_ref[h * HD:(h + 1) * HD, :],
            preferred_element_type=jnp.float32)

    hh = _ln(acc, s2_ref[...], t2_ref[...])
    a = jax.nn.gelu(
        jnp.dot(hh, w1_ref[...], preferred_element_type=jnp.float32)
        + b1_ref[...])
    o_ref[...] = acc + (
        jnp.dot(a, w2_ref[...], preferred_element_type=jnp.float32)
        + b2_ref[...])


def kernel(x, positions, c, Wq, bq, Wk, bk, Wv, bv, Wo, bo, W1, b1, W2, b2,
           ln1_scale, ln1_bias, ln2_scale, ln2_bias, log_tau, attn_scale):
    x0 = x[0]
    pos = positions[0]
    Wqkv = jnp.concatenate([Wq, Wk, Wv], axis=1)
    bqkv = jnp.concatenate([bq, bk, bv])[None, :]
    c2 = jnp.reshape(c, (1, 1)).astype(jnp.float32)
    lt2 = jnp.reshape(log_tau, (1, 1)).astype(jnp.float32)
    as2 = jnp.reshape(attn_scale, (1, 1)).astype(jnp.float32)

    qkv, arg = pl.pallas_call(
        _pre_body,
        grid=(GRID,),
        in_specs=[
            pl.BlockSpec((BLK, DIM), lambda i: (i, 0)),
            pl.BlockSpec((DIM, 3 * DIM), lambda i: (0, 0)),
            pl.BlockSpec((1, 3 * DIM), lambda i: (0, 0)),
            pl.BlockSpec((1, DIM), lambda i: (0, 0)),
            pl.BlockSpec((1, DIM), lambda i: (0, 0)),
            pl.BlockSpec((BLK, PDIM), lambda i: (i, 0)),
            pl.BlockSpec((N, PDIM), lambda i: (0, 0)),
            pl.BlockSpec((1, 1), lambda i: (0, 0)),
        ],
        out_specs=[
            pl.BlockSpec((BLK, 3 * DIM), lambda i: (i, 0)),
            pl.BlockSpec((BLK, N), lambda i: (i, 0)),
        ],
        out_shape=[
            jax.ShapeDtypeStruct((N, 3 * DIM), jnp.float32),
            jax.ShapeDtypeStruct((N, N), jnp.float32),
        ],
    )(x0, Wqkv, bqkv, ln1_scale[None, :], ln1_bias[None, :], pos, pos, c2)

    NH = N // 2
    GH = NH // BLK
    arg_flat = arg.reshape(N * N)
    halves = []
    for half in range(2):
        sc_topk = functools.partial(
            pl.kernel,
            mesh=plsc.VectorSubcoreMesh(core_axis_name="c",
                                        subcore_axis_name="s"),
            compiler_params=pltpu.CompilerParams(needs_layout_passes=False),
            out_type=[
                jax.ShapeDtypeStruct((NH * KNN,), jnp.int32),
                jax.ShapeDtypeStruct((NH * KNN,), jnp.float32),
            ],
            scratch_types=[
                pltpu.VMEM((N,), jnp.float32),
                pltpu.VMEM((KNN,), jnp.int32),
                pltpu.VMEM((KNN,), jnp.float32),
            ],
        )(functools.partial(_sc_topk, first_row=half * NH,
                            rows_per_worker=NH // NW))
        idx1d, val1d = sc_topk(arg_flat)
        halves.append((idx1d.reshape(NH, KNN), val1d.reshape(NH, KNN)))

    ys = []
    for half in range(2):
        idx, val = halves[half]
        off = half * GH
        y_h = pl.pallas_call(
            _attn_mlp_body,
            grid=(GH,),
            in_specs=[
                pl.BlockSpec((BLK, KNN), lambda i: (i, 0)),
                pl.BlockSpec((BLK, KNN), lambda i: (i, 0)),
                pl.BlockSpec((BLK, DIM), lambda i, o=off: (i + o, 0)),
                pl.BlockSpec((N, DIM), lambda i: (0, 1)),
                pl.BlockSpec((N, DIM), lambda i: (0, 2)),
                pl.BlockSpec((BLK, DIM), lambda i, o=off: (i + o, 0)),
                pl.BlockSpec((DIM, DIM), lambda i: (0, 0)),
                pl.BlockSpec((1, DIM), lambda i: (0, 0)),
                pl.BlockSpec((DIM, FF), lambda i: (0, 0)),
                pl.BlockSpec((1, FF), lambda i: (0, 0)),
                pl.BlockSpec((FF, DIM), lambda i: (0, 0)),
                pl.BlockSpec((1, DIM), lambda i: (0, 0)),
                pl.BlockSpec((1, DIM), lambda i: (0, 0)),
                pl.BlockSpec((1, DIM), lambda i: (0, 0)),
                pl.BlockSpec((1, 1), lambda i: (0, 0)),
                pl.BlockSpec((1, 1), lambda i: (0, 0)),
                pl.BlockSpec((1, 1), lambda i: (0, 0)),
            ],
            out_specs=pl.BlockSpec((BLK, DIM), lambda i: (i, 0)),
            out_shape=jax.ShapeDtypeStruct((NH, DIM), jnp.float32),
        )(idx, val, qkv, qkv, qkv, x0, Wo, bo[None, :], W1, b1[None, :],
          W2, b2[None, :], ln2_scale[None, :], ln2_bias[None, :],
          c2, lt2, as2)
        ys.append(y_h)

    return jnp.concatenate(ys, axis=0)[None]


# speedup vs baseline: 1.1636x; 1.1636x over previous
import functools
import math

import jax
import jax.numpy as jnp
from jax import lax
from jax.experimental import pallas as pl
from jax.experimental.pallas import tpu as pltpu
from jax.experimental.pallas import tpu_sc as plsc

N = 2048
DIM = 768
NHEADS = 12
HD = DIM // NHEADS
KNN = 16
PDIM = 16
FF = 4 * DIM
BLK = 256
GRID = N // BLK

NW = 32
RPW = N // NW


def _ln(x, scale, bias):
    m = jnp.mean(x, axis=-1, keepdims=True)
    xc = x - m
    v = jnp.mean(xc * xc, axis=-1, keepdims=True)
    return xc / jnp.sqrt(v + 1e-6) * scale + bias


def _pre_body(x_ref, w_ref, b_ref, s_ref, t_ref, pb_ref, pf_ref, c_ref,
              qkv_ref, arg_ref):
    xn = _ln(x_ref[...], s_ref[...], t_ref[...])
    qkv_ref[...] = (
        jnp.dot(xn, w_ref[...], preferred_element_type=jnp.float32) + b_ref[...]
    )

    c = c_ref[0, 0]
    pb = pb_ref[...]
    pf = pf_ref[...]
    nb = jnp.sum(pb * pb, axis=1, keepdims=True)
    ones_p = jnp.ones((1, PDIM), jnp.float32)
    nf = lax.dot_general(
        ones_p, pf * pf, (((1,), (1,)), ((), ())),
        precision=lax.Precision.HIGHEST,
        preferred_element_type=jnp.float32)
    g = lax.dot_general(
        pb, pf, (((1,), (1,)), ((), ())),
        precision=lax.Precision.HIGHEST,
        preferred_element_type=jnp.float32)
    diff = jnp.maximum(nb + nf - 2.0 * g, 0.0)
    den = jnp.maximum((1.0 - c * nb) * (1.0 - c * nf), 1e-8)
    arg_ref[...] = 1.0 + 2.0 * c * diff / den


def _sc_topk(arg_hbm, idx_hbm, val_hbm, row_v, i16_v, v16_v, *, first_row,
             rows_per_worker):
    wid = lax.axis_index("s") * 2 + lax.axis_index("c")
    base = first_row + wid * rows_per_worker
    lane = lax.iota(jnp.int32, 16)

    def _halfclean(a_k, a_i, b_k, b_i):
        m = b_k < a_k
        return jnp.where(m, b_k, a_k), jnp.where(m, b_i, a_i)

    def row_body(r, carry0):
        row = base + r
        pltpu.sync_copy(arg_hbm.at[row], row_v)

        def chunk_body(c8, carry):
            best, bidx = carry
            ks, vs = [], []
            for u in range(8):
                cc = c8 * 8 + u
                d = row_v[pl.ds(cc * 16, 16)]
                ci = lane + cc * 16
                sk, si = plsc.sort_key_val(d, ci, descending=(u % 2 == 1))
                ks.append(sk)
                vs.append(si)
            for _ in range(2):
                nk, nv = [], []
                for p in range(len(ks) // 2):
                    lk, li = _halfclean(ks[2 * p], vs[2 * p],
                                        ks[2 * p + 1], vs[2 * p + 1])
                    sk, si = plsc.sort_key_val(lk, li,
                                               descending=(p % 2 == 1))
                    nk.append(sk)
                    nv.append(si)
                ks, vs = nk, nv
            lk, li = _halfclean(ks[0], vs[0], ks[1], vs[1])
            xk, xi = plsc.sort_key_val(lk, li, descending=True)
            nb, ni = _halfclean(best, bidx, xk, xi)
            rk, ri = plsc.sort_key_val(nb, ni)
            return rk, ri

        best0 = jnp.full((16,), jnp.inf, jnp.float32)
        bidx0 = jnp.zeros((16,), jnp.int32)
        best, bidx = lax.fori_loop(0, N // 128, chunk_body, (best0, bidx0))
        i16_v[...] = bidx
        v16_v[...] = best
        out_off = (row - first_row) * KNN
        pltpu.sync_copy(i16_v, idx_hbm.at[pl.ds(out_off, KNN)])
        pltpu.sync_copy(v16_v, val_hbm.at[pl.ds(out_off, KNN)])
        return carry0

    lax.fori_loop(0, rows_per_worker, row_body, 0)


def _attn_mlp_body(idx_ref, val_ref, q_ref, k_ref, v_ref, x_ref, wo_ref,
                   bo_ref, w1_ref, b1_ref, w2_ref, b2_ref, s2_ref, t2_ref,
                   c_ref, lt_ref, as_ref, o_ref):
    c = c_ref[0, 0]
    inv_tau = 1.0 / jnp.maximum(jnp.exp(lt_ref[0, 0]), 1e-8)
    a_scale = as_ref[0, 0]
    inv_sqrt_c = 1.0 / jnp.sqrt(c)

    s = jnp.maximum(val_ref[...], 1.0 + 1e-7)
    d = jnp.log(s + jnp.sqrt((s - 1.0) * (s + 1.0))) * inv_sqrt_c
    gvals = -d * inv_tau
    idx = idx_ref[...]
    iota = lax.broadcasted_iota(jnp.int32, (BLK, N), 1)
    geo = jnp.zeros((BLK, N), jnp.float32)
    for t in range(KNN):
        geo = jnp.where(iota == idx[:, t:t + 1], gvals[:, t:t + 1], geo)
    mask = geo < 0.0

    inv_sqrt_hd = 1.0 / math.sqrt(HD)
    ones8 = jnp.ones((N, 8), jnp.float32)
    acc = x_ref[...] + bo_ref[...]
    for h in range(NHEADS):
        qh = q_ref[:, h * HD:(h + 1) * HD]
        kh = k_ref[:, h * HD:(h + 1) * HD]
        vh = v_ref[:, h * HD:(h + 1) * HD]
        sco = lax.dot_general(
            qh, kh, (((1,), (1,)), ((), ())),
            preferred_element_type=jnp.float32) * inv_sqrt_hd
        sc = a_scale * jnp.tanh(sco + geo)
        w = jnp.where(mask, jnp.exp(sc), 0.0)
        denom = jnp.dot(w, ones8, preferred_element_type=jnp.float32)
        oh = jnp.dot(w, vh, preferred_element_type=jnp.float32)
        oh = oh * (1.0 / denom[:, 0:1])
        acc = acc + jnp.dot(
            oh, wo_ref[h * HD:(h + 1) * HD, :],
            preferred_element_type=jnp.float32)

    hh = _ln(acc, s2_ref[...], t2_ref[...])
    a = jax.nn.gelu(
        jnp.dot(hh, w1_ref[...], preferred_element_type=jnp.float32)
        + b1_ref[...])
    o_ref[...] = acc + (
        jnp.dot(a, w2_ref[...], preferred_element_type=jnp.float32)
        + b2_ref[...])


def kernel(x, positions, c, Wq, bq, Wk, bk, Wv, bv, Wo, bo, W1, b1, W2, b2,
           ln1_scale, ln1_bias, ln2_scale, ln2_bias, log_tau, attn_scale):
    x0 = x[0]
    pos = positions[0]
    Wqkv = jnp.concatenate([Wq, Wk, Wv], axis=1)
    bqkv = jnp.concatenate([bq, bk, bv])[None, :]
    c2 = jnp.reshape(c, (1, 1)).astype(jnp.float32)
    lt2 = jnp.reshape(log_tau, (1, 1)).astype(jnp.float32)
    as2 = jnp.reshape(attn_scale, (1, 1)).astype(jnp.float32)

    qkv, arg = pl.pallas_call(
        _pre_body,
        grid=(GRID,),
        in_specs=[
            pl.BlockSpec((BLK, DIM), lambda i: (i, 0)),
            pl.BlockSpec((DIM, 3 * DIM), lambda i: (0, 0)),
            pl.BlockSpec((1, 3 * DIM), lambda i: (0, 0)),
            pl.BlockSpec((1, DIM), lambda i: (0, 0)),
            pl.BlockSpec((1, DIM), lambda i: (0, 0)),
            pl.BlockSpec((BLK, PDIM), lambda i: (i, 0)),
            pl.BlockSpec((N, PDIM), lambda i: (0, 0)),
            pl.BlockSpec((1, 1), lambda i: (0, 0)),
        ],
        out_specs=[
            pl.BlockSpec((BLK, 3 * DIM), lambda i: (i, 0)),
            pl.BlockSpec((BLK, N), lambda i: (i, 0)),
        ],
        out_shape=[
            jax.ShapeDtypeStruct((N, 3 * DIM), jnp.float32),
            jax.ShapeDtypeStruct((N, N), jnp.float32),
        ],
    )(x0, Wqkv, bqkv, ln1_scale[None, :], ln1_bias[None, :], pos, pos, c2)

    sc_topk = functools.partial(
        pl.kernel,
        mesh=plsc.VectorSubcoreMesh(core_axis_name="c", subcore_axis_name="s"),
        compiler_params=pltpu.CompilerParams(needs_layout_passes=False),
        out_type=[
            jax.ShapeDtypeStruct((N * KNN,), jnp.int32),
            jax.ShapeDtypeStruct((N * KNN,), jnp.float32),
        ],
        scratch_types=[
            pltpu.VMEM((N,), jnp.float32),
            pltpu.VMEM((KNN,), jnp.int32),
            pltpu.VMEM((KNN,), jnp.float32),
        ],
    )(functools.partial(_sc_topk, first_row=0, rows_per_worker=RPW))
    idx1d, val1d = sc_topk(arg)
    idx = idx1d.reshape(N, KNN)
    val = val1d.reshape(N, KNN)

    y = pl.pallas_call(
        _attn_mlp_body,
        grid=(GRID,),
        in_specs=[
            pl.BlockSpec((BLK, KNN), lambda i: (i, 0)),
            pl.BlockSpec((BLK, KNN), lambda i: (i, 0)),
            pl.BlockSpec((BLK, DIM), lambda i: (i, 0)),
            pl.BlockSpec((N, DIM), lambda i: (0, 1)),
            pl.BlockSpec((N, DIM), lambda i: (0, 2)),
            pl.BlockSpec((BLK, DIM), lambda i: (i, 0)),
            pl.BlockSpec((DIM, DIM), lambda i: (0, 0)),
            pl.BlockSpec((1, DIM), lambda i: (0, 0)),
            pl.BlockSpec((DIM, FF), lambda i: (0, 0)),
            pl.BlockSpec((1, FF), lambda i: (0, 0)),
            pl.BlockSpec((FF, DIM), lambda i: (0, 0)),
            pl.BlockSpec((1, DIM), lambda i: (0, 0)),
            pl.BlockSpec((1, DIM), lambda i: (0, 0)),
            pl.BlockSpec((1, DIM), lambda i: (0, 0)),
            pl.BlockSpec((1, 1), lambda i: (0, 0)),
            pl.BlockSpec((1, 1), lambda i: (0, 0)),
            pl.BlockSpec((1, 1), lambda i: (0, 0)),
        ],
        out_specs=pl.BlockSpec((BLK, DIM), lambda i: (i, 0)),
        out_shape=jax.ShapeDtypeStruct((N, DIM), jnp.float32),
    )(idx, val, qkv, qkv, qkv, x0, Wo, bo[None, :], W1, b1[None, :],
      W2, b2[None, :], ln2_scale[None, :], ln2_bias[None, :], c2, lt2, as2)

    return y[None]


# speedup vs baseline: 1.2797x; 1.0997x over previous
import functools
import math

import jax
import jax.numpy as jnp
from jax import lax
from jax.experimental import pallas as pl
from jax.experimental.pallas import tpu as pltpu
from jax.experimental.pallas import tpu_sc as plsc

N = 2048
DIM = 768
NHEADS = 12
HD = DIM // NHEADS
KNN = 16
PDIM = 16
FF = 4 * DIM
BLK = 256
GRID = N // BLK

NW = 32
RPW = N // NW


def _ln(x, scale, bias):
    m = jnp.mean(x, axis=-1, keepdims=True)
    xc = x - m
    v = jnp.mean(xc * xc, axis=-1, keepdims=True)
    return xc / jnp.sqrt(v + 1e-6) * scale + bias


def _pre_body(x_ref, w_ref, b_ref, s_ref, t_ref, pb_ref, pf_ref, c_ref,
              qkv_ref, arg_ref):
    xn = _ln(x_ref[...], s_ref[...], t_ref[...])
    qkv_ref[...] = (
        jnp.dot(xn, w_ref[...], preferred_element_type=jnp.float32) + b_ref[...]
    )

    c = c_ref[0, 0]
    pb = pb_ref[...]
    pf = pf_ref[...]
    nb = jnp.sum(pb * pb, axis=1, keepdims=True)
    ones_p = jnp.ones((1, PDIM), jnp.float32)
    nf = lax.dot_general(
        ones_p, pf * pf, (((1,), (1,)), ((), ())),
        precision=lax.Precision.HIGHEST,
        preferred_element_type=jnp.float32)
    g = lax.dot_general(
        pb, pf, (((1,), (1,)), ((), ())),
        precision=lax.Precision.HIGHEST,
        preferred_element_type=jnp.float32)
    diff = jnp.maximum(nb + nf - 2.0 * g, 0.0)
    den = jnp.maximum((1.0 - c * nb) * (1.0 - c * nf), 1e-8)
    arg_ref[...] = 1.0 + 2.0 * c * diff / den


def _sc_topk(arg_hbm, idx_hbm, val_hbm, rowa_v, rowb_v, i16_v, v16_v,
             sema, semb, *, first_row, rows_per_worker):
    wid = lax.axis_index("s") * 2 + lax.axis_index("c")
    base = first_row + wid * rows_per_worker
    lane = lax.iota(jnp.int32, 16)

    def _halfclean(a_k, a_i, b_k, b_i):
        m = b_k < a_k
        return jnp.where(m, b_k, a_k), jnp.where(m, b_i, a_i)

    def _process(row_v, row):
        def chunk_body(c8, carry):
            best, bidx = carry
            ks, vs = [], []
            for u in range(8):
                cc = c8 * 8 + u
                d = row_v[pl.ds(cc * 16, 16)]
                ci = lane + cc * 16
                sk, si = plsc.sort_key_val(d, ci, descending=(u % 2 == 1))
                ks.append(sk)
                vs.append(si)
            for _ in range(2):
                nk, nv = [], []
                for p in range(len(ks) // 2):
                    lk, li = _halfclean(ks[2 * p], vs[2 * p],
                                        ks[2 * p + 1], vs[2 * p + 1])
                    sk, si = plsc.sort_key_val(lk, li,
                                               descending=(p % 2 == 1))
                    nk.append(sk)
                    nv.append(si)
                ks, vs = nk, nv
            lk, li = _halfclean(ks[0], vs[0], ks[1], vs[1])
            xk, xi = plsc.sort_key_val(lk, li, descending=True)
            nb, ni = _halfclean(best, bidx, xk, xi)
            rk, ri = plsc.sort_key_val(nb, ni)
            return rk, ri

        best0 = jnp.full((16,), jnp.inf, jnp.float32)
        bidx0 = jnp.zeros((16,), jnp.int32)
        best, bidx = lax.fori_loop(0, N // 128, chunk_body, (best0, bidx0))
        i16_v[...] = bidx
        v16_v[...] = best
        out_off = (row - first_row) * KNN
        pltpu.sync_copy(i16_v, idx_hbm.at[pl.ds(out_off, KNN)])
        pltpu.sync_copy(v16_v, val_hbm.at[pl.ds(out_off, KNN)])

    pltpu.make_async_copy(arg_hbm.at[base], rowa_v, sema).start()

    def pair_body(r2, carry0):
        r = base + r2 * 2
        pltpu.make_async_copy(arg_hbm.at[r + 1], rowb_v, semb).start()
        pltpu.make_async_copy(arg_hbm.at[r], rowa_v, sema).wait()
        _process(rowa_v, r)
        nxt = jnp.minimum(r + 2, N - 1)
        pltpu.make_async_copy(arg_hbm.at[nxt], rowa_v, sema).start()
        pltpu.make_async_copy(arg_hbm.at[r + 1], rowb_v, semb).wait()
        _process(rowb_v, r + 1)
        return carry0

    lax.fori_loop(0, rows_per_worker // 2, pair_body, 0)
    pltpu.make_async_copy(arg_hbm.at[base], rowa_v, sema).wait()


def _attn_mlp_body(idx_ref, val_ref, q_ref, k_ref, v_ref, x_ref, wo_ref,
                   bo_ref, w1_ref, b1_ref, w2_ref, b2_ref, s2_ref, t2_ref,
                   c_ref, lt_ref, as_ref, o_ref):
    c = c_ref[0, 0]
    inv_tau = 1.0 / jnp.maximum(jnp.exp(lt_ref[0, 0]), 1e-8)
    a_scale = as_ref[0, 0]
    inv_sqrt_c = 1.0 / jnp.sqrt(c)

    s = jnp.maximum(val_ref[...], 1.0 + 1e-7)
    d = jnp.log(s + jnp.sqrt((s - 1.0) * (s + 1.0))) * inv_sqrt_c
    gvals = -d * inv_tau
    idx = idx_ref[...]
    iota = lax.broadcasted_iota(jnp.int32, (BLK, N), 1)
    geo = jnp.zeros((BLK, N), jnp.float32)
    for t in range(KNN):
        geo = jnp.where(iota == idx[:, t:t + 1], gvals[:, t:t + 1], geo)
    mask = geo < 0.0

    inv_sqrt_hd = 1.0 / math.sqrt(HD)
    ones8 = jnp.ones((N, 8), jnp.float32)
    acc = x_ref[...] + bo_ref[...]
    for h in range(NHEADS):
        qh = q_ref[:, h * HD:(h + 1) * HD]
        kh = k_ref[:, h * HD:(h + 1) * HD]
        vh = v_ref[:, h * HD:(h + 1) * HD]
        sco = lax.dot_general(
            qh, kh, (((1,), (1,)), ((), ())),
            preferred_element_type=jnp.float32) * inv_sqrt_hd
        sc = a_scale * jnp.tanh(sco + geo)
        w = jnp.where(mask, jnp.exp(sc), 0.0)
        denom = jnp.dot(w, ones8, preferred_element_type=jnp.float32)
        oh = jnp.dot(w, vh, preferred_element_type=jnp.float32)
        oh = oh * (1.0 / denom[:, 0:1])
        acc = acc + jnp.dot(
            oh, wo_ref[h * HD:(h + 1) * HD, :],
            preferred_element_type=jnp.float32)

    hh = _ln(acc, s2_ref[...], t2_ref[...])
    a = jax.nn.gelu(
        jnp.dot(hh, w1_ref[...], preferred_element_type=jnp.float32)
        + b1_ref[...])
    o_ref[...] = acc + (
        jnp.dot(a, w2_ref[...], preferred_element_type=jnp.float32)
        + b2_ref[...])


def kernel(x, positions, c, Wq, bq, Wk, bk, Wv, bv, Wo, bo, W1, b1, W2, b2,
           ln1_scale, ln1_bias, ln2_scale, ln2_bias, log_tau, attn_scale):
    x0 = x[0]
    pos = positions[0]
    Wqkv = jnp.concatenate([Wq, Wk, Wv], axis=1)
    bqkv = jnp.concatenate([bq, bk, bv])[None, :]
    c2 = jnp.reshape(c, (1, 1)).astype(jnp.float32)
    lt2 = jnp.reshape(log_tau, (1, 1)).astype(jnp.float32)
    as2 = jnp.reshape(attn_scale, (1, 1)).astype(jnp.float32)

    qkv, arg = pl.pallas_call(
        _pre_body,
        grid=(GRID,),
        in_specs=[
            pl.BlockSpec((BLK, DIM), lambda i: (i, 0)),
            pl.BlockSpec((DIM, 3 * DIM), lambda i: (0, 0)),
            pl.BlockSpec((1, 3 * DIM), lambda i: (0, 0)),
            pl.BlockSpec((1, DIM), lambda i: (0, 0)),
            pl.BlockSpec((1, DIM), lambda i: (0, 0)),
            pl.BlockSpec((BLK, PDIM), lambda i: (i, 0)),
            pl.BlockSpec((N, PDIM), lambda i: (0, 0)),
            pl.BlockSpec((1, 1), lambda i: (0, 0)),
        ],
        out_specs=[
            pl.BlockSpec((BLK, 3 * DIM), lambda i: (i, 0)),
            pl.BlockSpec((BLK, N), lambda i: (i, 0)),
        ],
        out_shape=[
            jax.ShapeDtypeStruct((N, 3 * DIM), jnp.float32),
            jax.ShapeDtypeStruct((N, N), jnp.float32),
        ],
    )(x0, Wqkv, bqkv, ln1_scale[None, :], ln1_bias[None, :], pos, pos, c2)

    sc_topk = functools.partial(
        pl.kernel,
        mesh=plsc.VectorSubcoreMesh(core_axis_name="c", subcore_axis_name="s"),
        compiler_params=pltpu.CompilerParams(needs_layout_passes=False),
        out_type=[
            jax.ShapeDtypeStruct((N * KNN,), jnp.int32),
            jax.ShapeDtypeStruct((N * KNN,), jnp.float32),
        ],
        scratch_types=[
            pltpu.VMEM((N,), jnp.float32),
            pltpu.VMEM((N,), jnp.float32),
            pltpu.VMEM((KNN,), jnp.int32),
            pltpu.VMEM((KNN,), jnp.float32),
            pltpu.SemaphoreType.DMA,
            pltpu.SemaphoreType.DMA,
        ],
    )(functools.partial(_sc_topk, first_row=0, rows_per_worker=RPW))
    idx1d, val1d = sc_topk(arg)
    idx = idx1d.reshape(N, KNN)
    val = val1d.reshape(N, KNN)

    y = pl.pallas_call(
        _attn_mlp_body,
        grid=(GRID,),
        in_specs=[
            pl.BlockSpec((BLK, KNN), lambda i: (i, 0)),
            pl.BlockSpec((BLK, KNN), lambda i: (i, 0)),
            pl.BlockSpec((BLK, DIM), lambda i: (i, 0)),
            pl.BlockSpec((N, DIM), lambda i: (0, 1)),
            pl.BlockSpec((N, DIM), lambda i: (0, 2)),
            pl.BlockSpec((BLK, DIM), lambda i: (i, 0)),
            pl.BlockSpec((DIM, DIM), lambda i: (0, 0)),
            pl.BlockSpec((1, DIM), lambda i: (0, 0)),
            pl.BlockSpec((DIM, FF), lambda i: (0, 0)),
            pl.BlockSpec((1, FF), lambda i: (0, 0)),
            pl.BlockSpec((FF, DIM), lambda i: (0, 0)),
            pl.BlockSpec((1, DIM), lambda i: (0, 0)),
            pl.BlockSpec((1, DIM), lambda i: (0, 0)),
            pl.BlockSpec((1, DIM), lambda i: (0, 0)),
            pl.BlockSpec((1, 1), lambda i: (0, 0)),
            pl.BlockSpec((1, 1), lambda i: (0, 0)),
            pl.BlockSpec((1, 1), lambda i: (0, 0)),
        ],
        out_specs=pl.BlockSpec((BLK, DIM), lambda i: (i, 0)),
        out_shape=jax.ShapeDtypeStruct((N, DIM), jnp.float32),
    )(idx, val, qkv, qkv, qkv, x0, Wo, bo[None, :], W1, b1[None, :],
      W2, b2[None, :], ln2_scale[None, :], ln2_bias[None, :], c2, lt2, as2)

    return y[None]
